# Initial kernel scaffold; baseline (speedup 1.0000x reference)
import jax
import jax.numpy as jnp
from jax.experimental import pallas as pl


def kernel(x, edge_index, gene_idx, W1, a1s, a1d, b1, W2, a2s, a2d, b2, W3, a3s, a3d, b3, Wc1, bc1, Wc2, bc2):
    raise NotImplementedError("write your pallas kernel here")


# speedup vs baseline: 1.0866x; 1.0866x over previous
import functools

import jax
import jax.numpy as jnp
from jax import lax
from jax.experimental import pallas as pl
from jax.experimental.pallas import tpu as pltpu


def _mm_body(x_ref, w_ref, b_ref, o_ref, *, act):
    acc = jnp.dot(x_ref[...], w_ref[...], preferred_element_type=jnp.float32)
    acc = acc + b_ref[...]
    if act == "elu":
        acc = jnp.where(acc > 0, acc, jnp.expm1(acc))
    elif act == "relu":
        acc = jnp.maximum(acc, 0.0)
    o_ref[...] = acc


def _mm(x, w, b, act="none", block_rows=2000):
    n, k = x.shape
    m = w.shape[1]
    if n % block_rows != 0:
        block_rows = n
    grid = (n // block_rows,)
    return pl.pallas_call(
        functools.partial(_mm_body, act=act),
        grid=grid,
        in_specs=[
            pl.BlockSpec((block_rows, k), lambda i: (i, 0)),
            pl.BlockSpec((k, m), lambda i: (0, 0)),
            pl.BlockSpec((1, m), lambda i: (0, 0)),
        ],
        out_specs=pl.BlockSpec((block_rows, m), lambda i: (i, 0)),
        out_shape=jax.ShapeDtypeStruct((n, m), jnp.float32),
    )(x, w, b.reshape(1, m))


def _gat_edges(xw, src, dst, a_s, a_d, heads, out_ch):
    n = xw.shape[0]
    xwh = xw.reshape(n, heads, out_ch)
    alpha_s = jnp.sum(xwh * a_s[None, :, :], axis=-1)
    alpha_d = jnp.sum(xwh * a_d[None, :, :], axis=-1)
    e = alpha_s[src] + alpha_d[dst]
    e = jnp.where(e > 0, e, 0.2 * e)
    ee = jnp.exp(e)
    denom = jax.ops.segment_sum(ee, dst, num_segments=n)
    alpha = ee / (denom[dst] + 1e-16)
    msg = xwh[src] * alpha[:, :, None]
    out = jax.ops.segment_sum(msg, dst, num_segments=n)
    return out.reshape(n, heads * out_ch)


def kernel(x, edge_index, gene_idx, W1, a1s, a1d, b1, W2, a2s, a2d, b2,
           W3, a3s, a3d, b3, Wc1, bc1, Wc2, bc2):
    n = x.shape[0]
    loop = jnp.arange(n, dtype=edge_index.dtype)
    src = jnp.concatenate([edge_index[0], loop])
    dst = jnp.concatenate([edge_index[1], loop])

    xw1 = _mm(x, W1, b1 * 0.0)
    h1 = _gat_edges(xw1, src, dst, a1s, a1d, 8, 64) + b1
    h1 = jnp.where(h1 > 0, h1, jnp.expm1(h1))

    xw2 = _mm(h1, W2, b2 * 0.0)
    h2 = _gat_edges(xw2, src, dst, a2s, a2d, 8, 64) + b2
    h2 = jnp.where(h2 > 0, h2, jnp.expm1(h2))

    xw3 = _mm(h2, W3, b3 * 0.0)
    h3 = _gat_edges(xw3, src, dst, a3s, a3d, 1, 64) + b3

    focal = h3[gene_idx]
    z = _mm(focal, Wc1, bc1, act="relu")
    return _mm(z, Wc2, bc2)


# speedup vs baseline: 13.8520x; 12.7484x over previous
import functools

import jax
import jax.numpy as jnp
from jax import lax
from jax.experimental import pallas as pl
from jax.experimental.pallas import tpu as pltpu
from jax.experimental.pallas import tpu_sc as plsc

NN = 10000
ETOT = 330000
EPAD = 331776
EPER12 = EPAD // 16
NCH12 = EPER12 // 128
EPER3 = EPAD // 32
NCH3 = EPER3 // 128
TPT = 632
NPAD = 16 * TPT
ACC12 = 144
ACC3 = 80

_i32 = jnp.int32
_f32 = jnp.float32

_SC_PARAMS = pltpu.CompilerParams(
    needs_layout_passes=False, use_tc_tiling_on_sc=False)


def _iota16():
    return lax.iota(_i32, 16)


def _c16(v):
    return jnp.full((16,), v, _i32)


def _sc12_body(xwf, tabsf, srcp, dstp, zer, out,
               src_c, dst_c, idxs, idxd, sbuf, dbuf, rows, msg,
               tmp, outb, acc, sem, sem2):
    cid = lax.axis_index("c")
    sid = lax.axis_index("s")
    ebase = sid * EPER12
    for r in range(16):
        msg[r, pl.ds(128, 16)] = jnp.zeros((16,), _f32)

    for gi in range(2):
        g = 2 * cid + gi
        goff = g * NN
        pltpu.sync_copy(zer, acc.at[pl.ds(pl.multiple_of(sid * TPT, 8), TPT)])
        plsc.subcore_barrier()

        def chunk(c, carry):
            cb = pl.multiple_of(ebase + c * 128, 8)
            pltpu.sync_copy(srcp.at[pl.ds(cb, 128)], src_c)
            pltpu.sync_copy(dstp.at[pl.ds(cb, 128)], dst_c)
            for k in range(8):
                s = pl.ds(k * 16, 16)
                idxs[s] = src_c[s] + goff
                idxd[s] = dst_c[s] + goff
            cps = pltpu.async_copy(tabsf.at[idxs], sbuf, sem2)
            cpd = pltpu.async_copy(tabsf.at[idxd], dbuf, sem2)
            cps.wait()
            cpd.wait()
            for b in range(8):
                o16 = b * 16
                lanes = o16 + _iota16()
                as0 = plsc.load_gather(sbuf, [lanes, _c16(0)])
                as1 = plsc.load_gather(sbuf, [lanes, _c16(1)])
                ad0 = plsc.load_gather(dbuf, [lanes, _c16(2)])
                ad1 = plsc.load_gather(dbuf, [lanes, _c16(3)])
                e0 = as0 + ad0
                e1 = as1 + ad1
                e0 = jnp.where(e0 > 0, e0, 0.2 * e0)
                e1 = jnp.where(e1 > 0, e1, 0.2 * e1)
                live = (cb + lanes) < ETOT
                ee0 = jnp.where(live, jnp.exp(e0), 0.0)
                ee1 = jnp.where(live, jnp.exp(e1), 0.0)
                plsc.store_scatter(msg, [_iota16(), _c16(128)], ee0)
                plsc.store_scatter(msg, [_iota16(), _c16(129)], ee1)
                rowvec = idxs[pl.ds(o16, 16)]
                dst_vec = dst_c[pl.ds(o16, 16)]
                pltpu.async_copy(xwf.at[rowvec], rows, sem).wait()
                for e in range(16):
                    bc0 = jnp.take_along_axis(ee0, _c16(e), axis=0)
                    bc1 = jnp.take_along_axis(ee1, _c16(e), axis=0)
                    for k in range(8):
                        v = rows[e, pl.ds(k * 16, 16)]
                        msg[e, pl.ds(k * 16, 16)] = v * (bc0 if k < 4 else bc1)
                pltpu.sync_copy(msg, acc.at[dst_vec], add=True)
            return carry

        lax.fori_loop(0, NCH12, chunk, 0)
        plsc.subcore_barrier()

        def dchunk(c, carry):
            r0 = pl.multiple_of(sid * TPT + c * 8, 8)
            pltpu.sync_copy(acc.at[pl.ds(r0, 8)], tmp)
            for r in range(8):
                d0 = plsc.load_gather(tmp, [_c16(r), _c16(128)])
                d1 = plsc.load_gather(tmp, [_c16(r), _c16(129)])
                inv0 = 1.0 / (d0 + 1e-16)
                inv1 = 1.0 / (d1 + 1e-16)
                for k in range(8):
                    v = tmp[r, pl.ds(k * 16, 16)]
                    outb[r, pl.ds(k * 16, 16)] = v * (inv0 if k < 4 else inv1)
            pltpu.sync_copy(outb, out.at[pl.ds(pl.multiple_of(goff + r0, 8), 8)])
            return carry

        nch = jnp.where(sid < 15, TPT // 8, (NN - 15 * TPT) // 8)
        lax.fori_loop(0, nch, dchunk, 0)
        plsc.subcore_barrier()


def _sc_layer12(xwf, tabsf, srcp, dstp, zer):
    mesh = plsc.VectorSubcoreMesh(core_axis_name="c", subcore_axis_name="s")
    return pl.kernel(
        _sc12_body,
        out_type=jax.ShapeDtypeStruct((4 * NN, 128), _f32),
        mesh=mesh,
        compiler_params=_SC_PARAMS,
        scratch_types=[
            pltpu.VMEM((128,), _i32),
            pltpu.VMEM((128,), _i32),
            pltpu.VMEM((128,), _i32),
            pltpu.VMEM((128,), _i32),
            pltpu.VMEM((128, 16), _f32),
            pltpu.VMEM((128, 16), _f32),
            pltpu.VMEM((16, 128), _f32),
            pltpu.VMEM((16, ACC12), _f32),
            pltpu.VMEM((8, ACC12), _f32),
            pltpu.VMEM((8, 128), _f32),
            pltpu.VMEM_SHARED((NPAD, ACC12), _f32),
            pltpu.SemaphoreType.DMA,
            pltpu.SemaphoreType.DMA,
        ],
    )(xwf, tabsf, srcp, dstp, zer)


def _sc3_body(xw3, tabsf, srcp, dstp, zer, out,
              src_c, dst_c, sbuf, dbuf, rows, msg, acc, sem, sem2):
    cid = lax.axis_index("c")
    sid = lax.axis_index("s")
    ebase = (cid * 16 + sid) * EPER3
    for r in range(16):
        msg[r, pl.ds(64, 16)] = jnp.zeros((16,), _f32)
    pltpu.sync_copy(zer, acc.at[pl.ds(pl.multiple_of(sid * TPT, 8), TPT)])
    plsc.subcore_barrier()

    def chunk(c, carry):
        cb = pl.multiple_of(ebase + c * 128, 8)
        pltpu.sync_copy(srcp.at[pl.ds(cb, 128)], src_c)
        pltpu.sync_copy(dstp.at[pl.ds(cb, 128)], dst_c)
        cps = pltpu.async_copy(tabsf.at[src_c], sbuf, sem2)
        cpd = pltpu.async_copy(tabsf.at[dst_c], dbuf, sem2)
        cps.wait()
        cpd.wait()
        for b in range(8):
            o16 = b * 16
            lanes = o16 + _iota16()
            e = (plsc.load_gather(sbuf, [lanes, _c16(0)]) +
                 plsc.load_gather(dbuf, [lanes, _c16(1)]))
            e = jnp.where(e > 0, e, 0.2 * e)
            live = (cb + lanes) < ETOT
            ee = jnp.where(live, jnp.exp(e), 0.0)
            plsc.store_scatter(msg, [_iota16(), _c16(64)], ee)
            src_vec = src_c[pl.ds(o16, 16)]
            dst_vec = dst_c[pl.ds(o16, 16)]
            pltpu.async_copy(xw3.at[src_vec], rows, sem).wait()
            for j in range(16):
                bc = jnp.take_along_axis(ee, _c16(j), axis=0)
                for k in range(4):
                    v = rows[j, pl.ds(k * 16, 16)]
                    msg[j, pl.ds(k * 16, 16)] = v * bc
            pltpu.sync_copy(msg, acc.at[dst_vec], add=True)
        return carry

    lax.fori_loop(0, NCH3, chunk, 0)
    plsc.subcore_barrier()

    @pl.when(sid < 15)
    def _():
        r0 = pl.multiple_of(sid * TPT, 8)
        pltpu.sync_copy(acc.at[pl.ds(r0, TPT)],
                        out.at[pl.ds(pl.multiple_of(cid * NN + r0, 8), TPT)])

    @pl.when(sid == 15)
    def _():
        rem = NN - 15 * TPT
        pltpu.sync_copy(acc.at[pl.ds(15 * TPT, rem)],
                        out.at[pl.ds(pl.multiple_of(cid * NN + 15 * TPT, 8), rem)])


def _sc_layer3(xw3, tabsf, srcp, dstp, zer):
    mesh = plsc.VectorSubcoreMesh(core_axis_name="c", subcore_axis_name="s")
    return pl.kernel(
        _sc3_body,
        out_type=jax.ShapeDtypeStruct((2 * NN, ACC3), _f32),
        mesh=mesh,
        compiler_params=_SC_PARAMS,
        scratch_types=[
            pltpu.VMEM((128,), _i32),
            pltpu.VMEM((128,), _i32),
            pltpu.VMEM((128, 16), _f32),
            pltpu.VMEM((128, 16), _f32),
            pltpu.VMEM((16, 64), _f32),
            pltpu.VMEM((16, ACC3), _f32),
            pltpu.VMEM_SHARED((NPAD, ACC3), _f32),
            pltpu.SemaphoreType.DMA,
            pltpu.SemaphoreType.DMA,
        ],
    )(xw3, tabsf, srcp, dstp, zer)


def _scf_body(p3, gidx, b3a, out, gv, gv2, r0, r1, fb, bv, sem):
    cid = lax.axis_index("c")
    sid = lax.axis_index("s")
    base = (cid * 16 + sid) * 32
    pltpu.sync_copy(gidx.at[pl.ds(pl.multiple_of(base, 8), 32)], gv)
    pltpu.sync_copy(b3a, bv)
    for k in range(2):
        gv2[pl.ds(k * 16, 16)] = gv[pl.ds(k * 16, 16)] + NN
    pltpu.async_copy(p3.at[gv], r0, sem).wait()
    pltpu.async_copy(p3.at[gv2], r1, sem).wait()
    for r in range(32):
        d = (plsc.load_gather(r0, [_c16(r), _c16(64)]) +
             plsc.load_gather(r1, [_c16(r), _c16(64)]))
        inv = 1.0 / (d + 1e-16)
        for k in range(4):
            s = pl.ds(k * 16, 16)
            fb[r, s] = (r0[r, s] + r1[r, s]) * inv + bv[s]
    pltpu.sync_copy(fb, out.at[pl.ds(pl.multiple_of(base, 8), 32)])


def _sc_focal(p3, gidx, b3):
    mesh = plsc.VectorSubcoreMesh(core_axis_name="c", subcore_axis_name="s")
    return pl.kernel(
        _scf_body,
        out_type=jax.ShapeDtypeStruct((1024, 64), _f32),
        mesh=mesh,
        compiler_params=_SC_PARAMS,
        scratch_types=[
            pltpu.VMEM((32,), _i32),
            pltpu.VMEM((32,), _i32),
            pltpu.VMEM((32, ACC3), _f32),
            pltpu.VMEM((32, ACC3), _f32),
            pltpu.VMEM((32, 64), _f32),
            pltpu.VMEM((64,), _f32),
            pltpu.SemaphoreType.DMA,
        ],
    )(p3, gidx, b3)


def _mmg_body(x_ref, w_ref, b_ref, o_ref, *, elu_in, gk_n):
    gk = pl.program_id(2)
    xb = x_ref[0]
    if elu_in:
        xb = xb + b_ref[0]
        xb = jnp.where(xb > 0, xb, jnp.exp(jnp.minimum(xb, 0.0)) - 1.0)
    p = jnp.dot(xb, w_ref[0], preferred_element_type=_f32)
    if gk_n == 1:
        o_ref[0] = p
    else:
        @pl.when(gk == 0)
        def _():
            o_ref[0] = p

        @pl.when(gk > 0)
        def _():
            o_ref[0] += p


def _mm_grouped(xg, wg, bin_, elu_in, g_out, wo, br=2000):
    gk, n, k = xg.shape
    grid = (n // br, g_out, gk)
    return pl.pallas_call(
        functools.partial(_mmg_body, elu_in=elu_in, gk_n=gk),
        grid=grid,
        in_specs=[
            pl.BlockSpec((1, br, k), lambda i, go, gki: (gki, i, 0)),
            pl.BlockSpec((1, k, wo), lambda i, go, gki: (gki, 0, go)),
            pl.BlockSpec((1, 1, k), lambda i, go, gki: (gki, 0, 0)),
        ],
        out_specs=pl.BlockSpec((1, br, wo), lambda i, go, gki: (go, i, 0)),
        out_shape=jax.ShapeDtypeStruct((g_out, n, wo), _f32),
    )(xg, wg, bin_)


def _tab_body(x_ref, a_ref, o_ref):
    o_ref[0] = jnp.dot(x_ref[0], a_ref[0], preferred_element_type=_f32)


def _tables4(xg, ag, br=2000):
    gk, n, k = xg.shape
    grid = (n // br, gk)
    return pl.pallas_call(
        _tab_body,
        grid=grid,
        in_specs=[
            pl.BlockSpec((1, br, k), lambda i, gki: (gki, i, 0)),
            pl.BlockSpec((1, k, 16), lambda i, gki: (gki, 0, 0)),
        ],
        out_specs=pl.BlockSpec((1, br, 16), lambda i, gki: (gki, i, 0)),
        out_shape=jax.ShapeDtypeStruct((gk, n, 16), _f32),
    )(xg, ag)


def _cls_body(f_ref, w1_ref, b1_ref, w2_ref, b2_ref, o_ref):
    z = jnp.dot(f_ref[...], w1_ref[...], preferred_element_type=_f32)
    z = jnp.maximum(z + b1_ref[...], 0.0)
    o_ref[...] = jnp.dot(z, w2_ref[...], preferred_element_type=_f32) + b2_ref[...]


def _classifier(focal, wc1, bc1, wc2, bc2):
    return pl.pallas_call(
        _cls_body,
        out_shape=jax.ShapeDtypeStruct((focal.shape[0], 2), _f32),
    )(focal, wc1, bc1.reshape(1, -1), wc2, bc2.reshape(1, -1))


def _attn_mat(a_s, a_d):
    a = jnp.zeros((4, 128, 16), _f32)
    for g in range(4):
        for j in range(2):
            hh = 2 * g + j
            a = a.at[g, j * 64:(j + 1) * 64, j].set(a_s[hh])
            a = a.at[g, j * 64:(j + 1) * 64, 2 + j].set(a_d[hh])
    return a


def kernel(x, edge_index, gene_idx, W1, a1s, a1d, b1, W2, a2s, a2d, b2,
           W3, a3s, a3d, b3, Wc1, bc1, Wc2, bc2):
    loop = jnp.arange(NN, dtype=edge_index.dtype)
    pad = jnp.zeros((EPAD - ETOT,), edge_index.dtype)
    srcp = jnp.concatenate([edge_index[0], loop, pad])
    dstp = jnp.concatenate([edge_index[1], loop, pad])
    zer12 = jnp.zeros((TPT, ACC12), _f32)
    zer3 = jnp.zeros((TPT, ACC3), _f32)
    zb = jnp.zeros((1, 1, 128), _f32)

    a1g = _attn_mat(a1s, a1d)
    a2g = _attn_mat(a2s, a2d)
    a3g = jnp.zeros((1, 64, 16), _f32).at[0, :, 0].set(a3s[0]).at[0, :, 1].set(a3d[0])

    xwr1 = _mm_grouped(x.reshape(1, NN, 128), W1.reshape(1, 128, 512),
                       zb, False, 4, 128)
    tabs1 = _tables4(xwr1, a1g)
    msg1 = _sc_layer12(xwr1.reshape(4 * NN, 128), tabs1.reshape(4 * NN, 16),
                       srcp, dstp, zer12)

    xwr2 = _mm_grouped(msg1.reshape(4, NN, 128), W2.reshape(4, 128, 512),
                       b1.reshape(4, 1, 128), True, 4, 128)
    tabs2 = _tables4(xwr2, a2g)
    msg2 = _sc_layer12(xwr2.reshape(4 * NN, 128), tabs2.reshape(4 * NN, 16),
                       srcp, dstp, zer12)

    xw3 = _mm_grouped(msg2.reshape(4, NN, 128), W3.reshape(4, 128, 64),
                      b2.reshape(4, 1, 128), True, 1, 64)
    xw3 = xw3.reshape(NN, 64)
    tabs3 = _tables4(xw3.reshape(1, NN, 64), a3g)
    p3 = _sc_layer3(xw3, tabs3.reshape(NN, 16), srcp, dstp, zer3)

    focal = _sc_focal(p3, gene_idx, b3)
    return _classifier(focal, Wc1, bc1, Wc2, bc2)


# speedup vs baseline: 15.4653x; 1.1165x over previous
import functools

import jax
import jax.numpy as jnp
from jax import lax
from jax.experimental import pallas as pl
from jax.experimental.pallas import tpu as pltpu
from jax.experimental.pallas import tpu_sc as plsc

NN = 10000
ETOT = 330000
EPAD = 331776
EPER12 = EPAD // 16
NCH12 = EPER12 // 128
EPER3 = EPAD // 32
NCH3 = EPER3 // 128
TPT = 632
NPAD = 16 * TPT
ACC12 = 144
ACC3 = 80

_i32 = jnp.int32
_f32 = jnp.float32

_SC_PARAMS = pltpu.CompilerParams(
    needs_layout_passes=False, use_tc_tiling_on_sc=False)


def _iota16():
    return lax.iota(_i32, 16)


def _c16(v):
    return jnp.full((16,), v, _i32)


def _sc12_body(xwf, tabsf, srcp, dstp, zer, out,
               src_c, dst_c, idxs, idxd, sbuf, dbuf, rows, msgc,
               acc, sem, sem2):
    cid = lax.axis_index("c")
    sid = lax.axis_index("s")
    ebase = sid * EPER12
    for r in range(128):
        msgc[r, pl.ds(128, 16)] = jnp.zeros((16,), _f32)

    for gi in range(2):
        g = 2 * cid + gi
        goff = g * NN
        pltpu.sync_copy(zer, acc.at[pl.ds(pl.multiple_of(sid * TPT, 8), TPT)])
        plsc.subcore_barrier()

        def chunk(c, carry):
            cb = pl.multiple_of(ebase + c * 128, 8)
            pltpu.sync_copy(srcp.at[pl.ds(cb, 128)], src_c)
            pltpu.sync_copy(dstp.at[pl.ds(cb, 128)], dst_c)
            for k in range(8):
                s = pl.ds(k * 16, 16)
                idxs[s] = src_c[s] + goff
                idxd[s] = dst_c[s] + goff
            cps = pltpu.async_copy(tabsf.at[idxs], sbuf, sem2)
            cpd = pltpu.async_copy(tabsf.at[idxd], dbuf, sem2)
            cpr = pltpu.async_copy(xwf.at[idxs], rows, sem)
            cps.wait()
            cpd.wait()
            cpr.wait()

            def batch(b, bc_):
                o16 = b * 16
                lanes = o16 + _iota16()
                as0 = plsc.load_gather(sbuf, [lanes, _c16(0)])
                as1 = plsc.load_gather(sbuf, [lanes, _c16(1)])
                ad0 = plsc.load_gather(dbuf, [lanes, _c16(2)])
                ad1 = plsc.load_gather(dbuf, [lanes, _c16(3)])
                e0 = as0 + ad0
                e1 = as1 + ad1
                e0 = jnp.where(e0 > 0, e0, 0.2 * e0)
                e1 = jnp.where(e1 > 0, e1, 0.2 * e1)
                live = (cb + lanes) < ETOT
                ee0 = jnp.where(live, jnp.exp(e0), 0.0)
                ee1 = jnp.where(live, jnp.exp(e1), 0.0)
                plsc.store_scatter(msgc, [lanes, _c16(128)], ee0)
                plsc.store_scatter(msgc, [lanes, _c16(129)], ee1)
                for e in range(16):
                    bc0 = jnp.take_along_axis(ee0, _c16(e), axis=0)
                    bc1 = jnp.take_along_axis(ee1, _c16(e), axis=0)
                    for k in range(8):
                        v = rows[o16 + e, pl.ds(k * 16, 16)]
                        msgc[o16 + e, pl.ds(k * 16, 16)] = (
                            v * (bc0 if k < 4 else bc1))
                return bc_

            lax.fori_loop(0, 8, batch, 0)
            pltpu.sync_copy(msgc, acc.at[dst_c], add=True)
            return carry

        lax.fori_loop(0, NCH12, chunk, 0)
        plsc.subcore_barrier()

        def dchunk(c, carry):
            r0 = pl.multiple_of(sid * TPT + c * 8, 8)
            pltpu.sync_copy(acc.at[pl.ds(r0, 8)], msgc.at[pl.ds(0, 8)])
            for r in range(8):
                d0 = plsc.load_gather(msgc, [_c16(r), _c16(128)])
                d1 = plsc.load_gather(msgc, [_c16(r), _c16(129)])
                inv0 = 1.0 / (d0 + 1e-16)
                inv1 = 1.0 / (d1 + 1e-16)
                for k in range(8):
                    v = msgc[r, pl.ds(k * 16, 16)]
                    rows[r, pl.ds(k * 16, 16)] = v * (inv0 if k < 4 else inv1)
            pltpu.sync_copy(rows.at[pl.ds(0, 8)],
                            out.at[pl.ds(pl.multiple_of(goff + r0, 8), 8)])
            return carry

        nch = jnp.where(sid < 15, TPT // 8, (NN - 15 * TPT) // 8)
        lax.fori_loop(0, nch, dchunk, 0)
        plsc.subcore_barrier()
        for r in range(8):
            msgc[r, pl.ds(128, 16)] = jnp.zeros((16,), _f32)


def _sc_layer12(xwf, tabsf, srcp, dstp, zer):
    mesh = plsc.VectorSubcoreMesh(core_axis_name="c", subcore_axis_name="s")
    return pl.kernel(
        _sc12_body,
        out_type=jax.ShapeDtypeStruct((4 * NN, 128), _f32),
        mesh=mesh,
        compiler_params=_SC_PARAMS,
        scratch_types=[
            pltpu.VMEM((128,), _i32),
            pltpu.VMEM((128,), _i32),
            pltpu.VMEM((128,), _i32),
            pltpu.VMEM((128,), _i32),
            pltpu.VMEM((128, 16), _f32),
            pltpu.VMEM((128, 16), _f32),
            pltpu.VMEM((128, 128), _f32),
            pltpu.VMEM((128, ACC12), _f32),
            pltpu.VMEM_SHARED((NPAD, ACC12), _f32),
            pltpu.SemaphoreType.DMA,
            pltpu.SemaphoreType.DMA,
        ],
    )(xwf, tabsf, srcp, dstp, zer)


def _sc3_body(xw3, tabsf, srcp, dstp, zer, out,
              src_c, dst_c, sbuf, dbuf, rows, msgc, acc, sem, sem2):
    cid = lax.axis_index("c")
    sid = lax.axis_index("s")
    ebase = (cid * 16 + sid) * EPER3
    for r in range(128):
        msgc[r, pl.ds(64, 16)] = jnp.zeros((16,), _f32)
    pltpu.sync_copy(zer, acc.at[pl.ds(pl.multiple_of(sid * TPT, 8), TPT)])
    plsc.subcore_barrier()

    def chunk(c, carry):
        cb = pl.multiple_of(ebase + c * 128, 8)
        pltpu.sync_copy(srcp.at[pl.ds(cb, 128)], src_c)
        pltpu.sync_copy(dstp.at[pl.ds(cb, 128)], dst_c)
        cps = pltpu.async_copy(tabsf.at[src_c], sbuf, sem2)
        cpd = pltpu.async_copy(tabsf.at[dst_c], dbuf, sem2)
        cpr = pltpu.async_copy(xw3.at[src_c], rows, sem)
        cps.wait()
        cpd.wait()
        cpr.wait()

        def batch(b, bc_):
            o16 = b * 16
            lanes = o16 + _iota16()
            e = (plsc.load_gather(sbuf, [lanes, _c16(0)]) +
                 plsc.load_gather(dbuf, [lanes, _c16(1)]))
            e = jnp.where(e > 0, e, 0.2 * e)
            live = (cb + lanes) < ETOT
            ee = jnp.where(live, jnp.exp(e), 0.0)
            plsc.store_scatter(msgc, [lanes, _c16(64)], ee)
            for j in range(16):
                bc = jnp.take_along_axis(ee, _c16(j), axis=0)
                for k in range(4):
                    v = rows[o16 + j, pl.ds(k * 16, 16)]
                    msgc[o16 + j, pl.ds(k * 16, 16)] = v * bc
            return bc_

        lax.fori_loop(0, 8, batch, 0)
        pltpu.sync_copy(msgc, acc.at[dst_c], add=True)
        return carry

    lax.fori_loop(0, NCH3, chunk, 0)
    plsc.subcore_barrier()

    @pl.when(sid < 15)
    def _():
        r0 = pl.multiple_of(sid * TPT, 8)
        pltpu.sync_copy(acc.at[pl.ds(r0, TPT)],
                        out.at[pl.ds(pl.multiple_of(cid * NN + r0, 8), TPT)])

    @pl.when(sid == 15)
    def _():
        rem = NN - 15 * TPT
        pltpu.sync_copy(acc.at[pl.ds(15 * TPT, rem)],
                        out.at[pl.ds(pl.multiple_of(cid * NN + 15 * TPT, 8), rem)])


def _sc_layer3(xw3, tabsf, srcp, dstp, zer):
    mesh = plsc.VectorSubcoreMesh(core_axis_name="c", subcore_axis_name="s")
    return pl.kernel(
        _sc3_body,
        out_type=jax.ShapeDtypeStruct((2 * NN, ACC3), _f32),
        mesh=mesh,
        compiler_params=_SC_PARAMS,
        scratch_types=[
            pltpu.VMEM((128,), _i32),
            pltpu.VMEM((128,), _i32),
            pltpu.VMEM((128, 16), _f32),
            pltpu.VMEM((128, 16), _f32),
            pltpu.VMEM((128, 64), _f32),
            pltpu.VMEM((128, ACC3), _f32),
            pltpu.VMEM_SHARED((NPAD, ACC3), _f32),
            pltpu.SemaphoreType.DMA,
            pltpu.SemaphoreType.DMA,
        ],
    )(xw3, tabsf, srcp, dstp, zer)


def _scf_body(p3, gidx, b3a, out, gv, gv2, r0, r1, fb, bv, sem):
    cid = lax.axis_index("c")
    sid = lax.axis_index("s")
    base = (cid * 16 + sid) * 32
    pltpu.sync_copy(gidx.at[pl.ds(pl.multiple_of(base, 8), 32)], gv)
    pltpu.sync_copy(b3a, bv)
    for k in range(2):
        gv2[pl.ds(k * 16, 16)] = gv[pl.ds(k * 16, 16)] + NN
    pltpu.async_copy(p3.at[gv], r0, sem).wait()
    pltpu.async_copy(p3.at[gv2], r1, sem).wait()
    for r in range(32):
        d = (plsc.load_gather(r0, [_c16(r), _c16(64)]) +
             plsc.load_gather(r1, [_c16(r), _c16(64)]))
        inv = 1.0 / (d + 1e-16)
        for k in range(4):
            s = pl.ds(k * 16, 16)
            fb[r, s] = (r0[r, s] + r1[r, s]) * inv + bv[s]
    pltpu.sync_copy(fb, out.at[pl.ds(pl.multiple_of(base, 8), 32)])


def _sc_focal(p3, gidx, b3):
    mesh = plsc.VectorSubcoreMesh(core_axis_name="c", subcore_axis_name="s")
    return pl.kernel(
        _scf_body,
        out_type=jax.ShapeDtypeStruct((1024, 64), _f32),
        mesh=mesh,
        compiler_params=_SC_PARAMS,
        scratch_types=[
            pltpu.VMEM((32,), _i32),
            pltpu.VMEM((32,), _i32),
            pltpu.VMEM((32, ACC3), _f32),
            pltpu.VMEM((32, ACC3), _f32),
            pltpu.VMEM((32, 64), _f32),
            pltpu.VMEM((64,), _f32),
            pltpu.SemaphoreType.DMA,
        ],
    )(p3, gidx, b3)


def _mmg_body(x_ref, w_ref, b_ref, o_ref, *, elu_in, gk_n):
    gk = pl.program_id(2)
    xb = x_ref[0]
    if elu_in:
        xb = xb + b_ref[0]
        xb = jnp.where(xb > 0, xb, jnp.exp(jnp.minimum(xb, 0.0)) - 1.0)
    p = jnp.dot(xb, w_ref[0], preferred_element_type=_f32)
    if gk_n == 1:
        o_ref[0] = p
    else:
        @pl.when(gk == 0)
        def _():
            o_ref[0] = p

        @pl.when(gk > 0)
        def _():
            o_ref[0] += p


def _mm_grouped(xg, wg, bin_, elu_in, g_out, wo, br=2000):
    gk, n, k = xg.shape
    grid = (n // br, g_out, gk)
    return pl.pallas_call(
        functools.partial(_mmg_body, elu_in=elu_in, gk_n=gk),
        grid=grid,
        in_specs=[
            pl.BlockSpec((1, br, k), lambda i, go, gki: (gki, i, 0)),
            pl.BlockSpec((1, k, wo), lambda i, go, gki: (gki, 0, go)),
            pl.BlockSpec((1, 1, k), lambda i, go, gki: (gki, 0, 0)),
        ],
        out_specs=pl.BlockSpec((1, br, wo), lambda i, go, gki: (go, i, 0)),
        out_shape=jax.ShapeDtypeStruct((g_out, n, wo), _f32),
    )(xg, wg, bin_)


def _tab_body(x_ref, a_ref, o_ref):
    o_ref[0] = jnp.dot(x_ref[0], a_ref[0], preferred_element_type=_f32)


def _tables4(xg, ag, br=2000):
    gk, n, k = xg.shape
    grid = (n // br, gk)
    return pl.pallas_call(
        _tab_body,
        grid=grid,
        in_specs=[
            pl.BlockSpec((1, br, k), lambda i, gki: (gki, i, 0)),
            pl.BlockSpec((1, k, 16), lambda i, gki: (gki, 0, 0)),
        ],
        out_specs=pl.BlockSpec((1, br, 16), lambda i, gki: (gki, i, 0)),
        out_shape=jax.ShapeDtypeStruct((gk, n, 16), _f32),
    )(xg, ag)


def _cls_body(f_ref, w1_ref, b1_ref, w2_ref, b2_ref, o_ref):
    z = jnp.dot(f_ref[...], w1_ref[...], preferred_element_type=_f32)
    z = jnp.maximum(z + b1_ref[...], 0.0)
    o_ref[...] = jnp.dot(z, w2_ref[...], preferred_element_type=_f32) + b2_ref[...]


def _classifier(focal, wc1, bc1, wc2, bc2):
    return pl.pallas_call(
        _cls_body,
        out_shape=jax.ShapeDtypeStruct((focal.shape[0], 2), _f32),
    )(focal, wc1, bc1.reshape(1, -1), wc2, bc2.reshape(1, -1))


def _attn_mat(a_s, a_d):
    a = jnp.zeros((4, 128, 16), _f32)
    for g in range(4):
        for j in range(2):
            hh = 2 * g + j
            a = a.at[g, j * 64:(j + 1) * 64, j].set(a_s[hh])
            a = a.at[g, j * 64:(j + 1) * 64, 2 + j].set(a_d[hh])
    return a


def kernel(x, edge_index, gene_idx, W1, a1s, a1d, b1, W2, a2s, a2d, b2,
           W3, a3s, a3d, b3, Wc1, bc1, Wc2, bc2):
    loop = jnp.arange(NN, dtype=edge_index.dtype)
    pad = jnp.zeros((EPAD - ETOT,), edge_index.dtype)
    srcp = jnp.concatenate([edge_index[0], loop, pad])
    dstp = jnp.concatenate([edge_index[1], loop, pad])
    zer12 = jnp.zeros((TPT, ACC12), _f32)
    zer3 = jnp.zeros((TPT, ACC3), _f32)
    zb = jnp.zeros((1, 1, 128), _f32)

    a1g = _attn_mat(a1s, a1d)
    a2g = _attn_mat(a2s, a2d)
    a3g = jnp.zeros((1, 64, 16), _f32).at[0, :, 0].set(a3s[0]).at[0, :, 1].set(a3d[0])

    xwr1 = _mm_grouped(x.reshape(1, NN, 128), W1.reshape(1, 128, 512),
                       zb, False, 4, 128)
    tabs1 = _tables4(xwr1, a1g)
    msg1 = _sc_layer12(xwr1.reshape(4 * NN, 128), tabs1.reshape(4 * NN, 16),
                       srcp, dstp, zer12)

    xwr2 = _mm_grouped(msg1.reshape(4, NN, 128), W2.reshape(4, 128, 512),
                       b1.reshape(4, 1, 128), True, 4, 128)
    tabs2 = _tables4(xwr2, a2g)
    msg2 = _sc_layer12(xwr2.reshape(4 * NN, 128), tabs2.reshape(4 * NN, 16),
                       srcp, dstp, zer12)

    xw3 = _mm_grouped(msg2.reshape(4, NN, 128), W3.reshape(4, 128, 64),
                      b2.reshape(4, 1, 128), True, 1, 64)
    xw3 = xw3.reshape(NN, 64)
    tabs3 = _tables4(xw3.reshape(1, NN, 64), a3g)
    p3 = _sc_layer3(xw3, tabs3.reshape(NN, 16), srcp, dstp, zer3)

    focal = _sc_focal(p3, gene_idx, b3)
    return _classifier(focal, Wc1, bc1, Wc2, bc2)


# speedup vs baseline: 16.7164x; 1.0809x over previous
import functools

import jax
import jax.numpy as jnp
from jax import lax
from jax.experimental import pallas as pl
from jax.experimental.pallas import tpu as pltpu
from jax.experimental.pallas import tpu_sc as plsc

NN = 10000
ETOT = 330000
EPAD = 331776
EPER12 = EPAD // 16
NCH12 = EPER12 // 128
EPER3 = EPAD // 32
NCH3 = EPER3 // 128
TPT = 632
NPAD = 16 * TPT
ACC12 = 144
ACC3 = 80

_i32 = jnp.int32
_f32 = jnp.float32

_SC_PARAMS = pltpu.CompilerParams(
    needs_layout_passes=False, use_tc_tiling_on_sc=False)


def _iota16():
    return lax.iota(_i32, 16)


def _c16(v):
    return jnp.full((16,), v, _i32)


def _sc12_body(xwf, tabsf, srcp, dstp, zer, out,
               src_c, dst_c, dst_p, idxs, idxd, sbuf, dbuf, rows, msgc,
               acc, sem, sem2, sem3):
    cid = lax.axis_index("c")
    sid = lax.axis_index("s")
    ebase = sid * EPER12
    for r in range(128):
        msgc[r, pl.ds(128, 16)] = jnp.zeros((16,), _f32)

    for gi in range(2):
        g = 2 * cid + gi
        goff = g * NN
        pltpu.sync_copy(zer, acc.at[pl.ds(pl.multiple_of(sid * TPT, 8), TPT)])
        plsc.subcore_barrier()

        def chunk(c, carry):
            cb = pl.multiple_of(ebase + c * 128, 8)
            pltpu.sync_copy(srcp.at[pl.ds(cb, 128)], src_c)
            pltpu.sync_copy(dstp.at[pl.ds(cb, 128)], dst_c)
            for k in range(8):
                s = pl.ds(k * 16, 16)
                idxs[s] = src_c[s] + goff
                idxd[s] = dst_c[s] + goff
            cps = pltpu.async_copy(tabsf.at[idxs], sbuf, sem2)
            cpd = pltpu.async_copy(tabsf.at[idxd], dbuf, sem2)
            cpr = pltpu.async_copy(xwf.at[idxs], rows, sem)

            @pl.when(c > 0)
            def _():
                pltpu.make_async_copy(zer.at[pl.ds(0, 128)], msgc, sem3).wait()
            cps.wait()
            cpd.wait()
            cpr.wait()

            def batch(b, bc_):
                o16 = b * 16
                lanes = o16 + _iota16()
                as0 = plsc.load_gather(sbuf, [lanes, _c16(0)])
                as1 = plsc.load_gather(sbuf, [lanes, _c16(1)])
                ad0 = plsc.load_gather(dbuf, [lanes, _c16(2)])
                ad1 = plsc.load_gather(dbuf, [lanes, _c16(3)])
                e0 = as0 + ad0
                e1 = as1 + ad1
                e0 = jnp.where(e0 > 0, e0, 0.2 * e0)
                e1 = jnp.where(e1 > 0, e1, 0.2 * e1)
                live = (cb + lanes) < ETOT
                ee0 = jnp.where(live, jnp.exp(e0), 0.0)
                ee1 = jnp.where(live, jnp.exp(e1), 0.0)
                plsc.store_scatter(msgc, [lanes, _c16(128)], ee0)
                plsc.store_scatter(msgc, [lanes, _c16(129)], ee1)
                for e in range(16):
                    bc0 = jnp.take_along_axis(ee0, _c16(e), axis=0)
                    bc1 = jnp.take_along_axis(ee1, _c16(e), axis=0)
                    for k in range(8):
                        v = rows[o16 + e, pl.ds(k * 16, 16)]
                        msgc[o16 + e, pl.ds(k * 16, 16)] = (
                            v * (bc0 if k < 4 else bc1))
                return bc_

            lax.fori_loop(0, 8, batch, 0)
            for k in range(8):
                s_ = pl.ds(k * 16, 16)
                dst_p[s_] = dst_c[s_]
            pltpu.async_copy(msgc, acc.at[dst_p], sem3, add=True)
            return carry

        lax.fori_loop(0, NCH12, chunk, 0)
        pltpu.make_async_copy(zer.at[pl.ds(0, 128)], msgc, sem3).wait()
        plsc.subcore_barrier()

        def dchunk(c, carry):
            r0 = pl.multiple_of(sid * TPT + c * 8, 8)
            pltpu.sync_copy(acc.at[pl.ds(r0, 8)], msgc.at[pl.ds(0, 8)])
            for r in range(8):
                d0 = plsc.load_gather(msgc, [_c16(r), _c16(128)])
                d1 = plsc.load_gather(msgc, [_c16(r), _c16(129)])
                inv0 = 1.0 / (d0 + 1e-16)
                inv1 = 1.0 / (d1 + 1e-16)
                for k in range(8):
                    v = msgc[r, pl.ds(k * 16, 16)]
                    rows[r, pl.ds(k * 16, 16)] = v * (inv0 if k < 4 else inv1)
            pltpu.sync_copy(rows.at[pl.ds(0, 8)],
                            out.at[pl.ds(pl.multiple_of(goff + r0, 8), 8)])
            return carry

        nch = jnp.where(sid < 15, TPT // 8, (NN - 15 * TPT) // 8)
        lax.fori_loop(0, nch, dchunk, 0)
        plsc.subcore_barrier()
        for r in range(8):
            msgc[r, pl.ds(128, 16)] = jnp.zeros((16,), _f32)


def _sc_layer12(xwf, tabsf, srcp, dstp, zer):
    mesh = plsc.VectorSubcoreMesh(core_axis_name="c", subcore_axis_name="s")
    return pl.kernel(
        _sc12_body,
        out_type=jax.ShapeDtypeStruct((4 * NN, 128), _f32),
        mesh=mesh,
        compiler_params=_SC_PARAMS,
        scratch_types=[
            pltpu.VMEM((128,), _i32),
            pltpu.VMEM((128,), _i32),
            pltpu.VMEM((128,), _i32),
            pltpu.VMEM((128,), _i32),
            pltpu.VMEM((128,), _i32),
            pltpu.VMEM((128, 16), _f32),
            pltpu.VMEM((128, 16), _f32),
            pltpu.VMEM((128, 128), _f32),
            pltpu.VMEM((128, ACC12), _f32),
            pltpu.VMEM_SHARED((NPAD, ACC12), _f32),
            pltpu.SemaphoreType.DMA,
            pltpu.SemaphoreType.DMA,
            pltpu.SemaphoreType.DMA,
        ],
    )(xwf, tabsf, srcp, dstp, zer)


def _sc3_body(xw3, tabsf, srcp, dstp, zer, out,
              src_c, dst_c, dst_p, sbuf, dbuf, rows, msgc, acc, sem, sem2, sem3):
    cid = lax.axis_index("c")
    sid = lax.axis_index("s")
    ebase = (cid * 16 + sid) * EPER3
    for r in range(128):
        msgc[r, pl.ds(64, 16)] = jnp.zeros((16,), _f32)
    pltpu.sync_copy(zer, acc.at[pl.ds(pl.multiple_of(sid * TPT, 8), TPT)])
    plsc.subcore_barrier()

    def chunk(c, carry):
        cb = pl.multiple_of(ebase + c * 128, 8)
        pltpu.sync_copy(srcp.at[pl.ds(cb, 128)], src_c)
        pltpu.sync_copy(dstp.at[pl.ds(cb, 128)], dst_c)
        cps = pltpu.async_copy(tabsf.at[src_c], sbuf, sem2)
        cpd = pltpu.async_copy(tabsf.at[dst_c], dbuf, sem2)
        cpr = pltpu.async_copy(xw3.at[src_c], rows, sem)

        @pl.when(c > 0)
        def _():
            pltpu.make_async_copy(zer.at[pl.ds(0, 128)], msgc, sem3).wait()
        cps.wait()
        cpd.wait()
        cpr.wait()

        def batch(b, bc_):
            o16 = b * 16
            lanes = o16 + _iota16()
            e = (plsc.load_gather(sbuf, [lanes, _c16(0)]) +
                 plsc.load_gather(dbuf, [lanes, _c16(1)]))
            e = jnp.where(e > 0, e, 0.2 * e)
            live = (cb + lanes) < ETOT
            ee = jnp.where(live, jnp.exp(e), 0.0)
            plsc.store_scatter(msgc, [lanes, _c16(64)], ee)
            for j in range(16):
                bc = jnp.take_along_axis(ee, _c16(j), axis=0)
                for k in range(4):
                    v = rows[o16 + j, pl.ds(k * 16, 16)]
                    msgc[o16 + j, pl.ds(k * 16, 16)] = v * bc
            return bc_

        lax.fori_loop(0, 8, batch, 0)
        for k in range(8):
            s_ = pl.ds(k * 16, 16)
            dst_p[s_] = dst_c[s_]
        pltpu.async_copy(msgc, acc.at[dst_p], sem3, add=True)
        return carry

    lax.fori_loop(0, NCH3, chunk, 0)
    pltpu.make_async_copy(zer.at[pl.ds(0, 128)], msgc, sem3).wait()
    plsc.subcore_barrier()

    @pl.when(sid < 15)
    def _():
        r0 = pl.multiple_of(sid * TPT, 8)
        pltpu.sync_copy(acc.at[pl.ds(r0, TPT)],
                        out.at[pl.ds(pl.multiple_of(cid * NN + r0, 8), TPT)])

    @pl.when(sid == 15)
    def _():
        rem = NN - 15 * TPT
        pltpu.sync_copy(acc.at[pl.ds(15 * TPT, rem)],
                        out.at[pl.ds(pl.multiple_of(cid * NN + 15 * TPT, 8), rem)])


def _sc_layer3(xw3, tabsf, srcp, dstp, zer):
    mesh = plsc.VectorSubcoreMesh(core_axis_name="c", subcore_axis_name="s")
    return pl.kernel(
        _sc3_body,
        out_type=jax.ShapeDtypeStruct((2 * NN, ACC3), _f32),
        mesh=mesh,
        compiler_params=_SC_PARAMS,
        scratch_types=[
            pltpu.VMEM((128,), _i32),
            pltpu.VMEM((128,), _i32),
            pltpu.VMEM((128,), _i32),
            pltpu.VMEM((128, 16), _f32),
            pltpu.VMEM((128, 16), _f32),
            pltpu.VMEM((128, 64), _f32),
            pltpu.VMEM((128, ACC3), _f32),
            pltpu.VMEM_SHARED((NPAD, ACC3), _f32),
            pltpu.SemaphoreType.DMA,
            pltpu.SemaphoreType.DMA,
            pltpu.SemaphoreType.DMA,
        ],
    )(xw3, tabsf, srcp, dstp, zer)


def _scf_body(p3, gidx, b3a, out, gv, gv2, r0, r1, fb, bv, sem):
    cid = lax.axis_index("c")
    sid = lax.axis_index("s")
    base = (cid * 16 + sid) * 32
    pltpu.sync_copy(gidx.at[pl.ds(pl.multiple_of(base, 8), 32)], gv)
    pltpu.sync_copy(b3a, bv)
    for k in range(2):
        gv2[pl.ds(k * 16, 16)] = gv[pl.ds(k * 16, 16)] + NN
    pltpu.async_copy(p3.at[gv], r0, sem).wait()
    pltpu.async_copy(p3.at[gv2], r1, sem).wait()
    for r in range(32):
        d = (plsc.load_gather(r0, [_c16(r), _c16(64)]) +
             plsc.load_gather(r1, [_c16(r), _c16(64)]))
        inv = 1.0 / (d + 1e-16)
        for k in range(4):
            s = pl.ds(k * 16, 16)
            fb[r, s] = (r0[r, s] + r1[r, s]) * inv + bv[s]
    pltpu.sync_copy(fb, out.at[pl.ds(pl.multiple_of(base, 8), 32)])


def _sc_focal(p3, gidx, b3):
    mesh = plsc.VectorSubcoreMesh(core_axis_name="c", subcore_axis_name="s")
    return pl.kernel(
        _scf_body,
        out_type=jax.ShapeDtypeStruct((1024, 64), _f32),
        mesh=mesh,
        compiler_params=_SC_PARAMS,
        scratch_types=[
            pltpu.VMEM((32,), _i32),
            pltpu.VMEM((32,), _i32),
            pltpu.VMEM((32, ACC3), _f32),
            pltpu.VMEM((32, ACC3), _f32),
            pltpu.VMEM((32, 64), _f32),
            pltpu.VMEM((64,), _f32),
            pltpu.SemaphoreType.DMA,
        ],
    )(p3, gidx, b3)


def _mmg_body(x_ref, w_ref, b_ref, o_ref, *, elu_in, gk_n):
    gk = pl.program_id(2)
    xb = x_ref[0]
    if elu_in:
        xb = xb + b_ref[0]
        xb = jnp.where(xb > 0, xb, jnp.exp(jnp.minimum(xb, 0.0)) - 1.0)
    p = jnp.dot(xb, w_ref[0], preferred_element_type=_f32)
    if gk_n == 1:
        o_ref[0] = p
    else:
        @pl.when(gk == 0)
        def _():
            o_ref[0] = p

        @pl.when(gk > 0)
        def _():
            o_ref[0] += p


def _mm_grouped(xg, wg, bin_, elu_in, g_out, wo, br=2000):
    gk, n, k = xg.shape
    grid = (n // br, g_out, gk)
    return pl.pallas_call(
        functools.partial(_mmg_body, elu_in=elu_in, gk_n=gk),
        grid=grid,
        in_specs=[
            pl.BlockSpec((1, br, k), lambda i, go, gki: (gki, i, 0)),
            pl.BlockSpec((1, k, wo), lambda i, go, gki: (gki, 0, go)),
            pl.BlockSpec((1, 1, k), lambda i, go, gki: (gki, 0, 0)),
        ],
        out_specs=pl.BlockSpec((1, br, wo), lambda i, go, gki: (go, i, 0)),
        out_shape=jax.ShapeDtypeStruct((g_out, n, wo), _f32),
    )(xg, wg, bin_)


def _tab_body(x_ref, a_ref, o_ref):
    o_ref[0] = jnp.dot(x_ref[0], a_ref[0], preferred_element_type=_f32)


def _tables4(xg, ag, br=2000):
    gk, n, k = xg.shape
    grid = (n // br, gk)
    return pl.pallas_call(
        _tab_body,
        grid=grid,
        in_specs=[
            pl.BlockSpec((1, br, k), lambda i, gki: (gki, i, 0)),
            pl.BlockSpec((1, k, 16), lambda i, gki: (gki, 0, 0)),
        ],
        out_specs=pl.BlockSpec((1, br, 16), lambda i, gki: (gki, i, 0)),
        out_shape=jax.ShapeDtypeStruct((gk, n, 16), _f32),
    )(xg, ag)


def _cls_body(f_ref, w1_ref, b1_ref, w2_ref, b2_ref, o_ref):
    z = jnp.dot(f_ref[...], w1_ref[...], preferred_element_type=_f32)
    z = jnp.maximum(z + b1_ref[...], 0.0)
    o_ref[...] = jnp.dot(z, w2_ref[...], preferred_element_type=_f32) + b2_ref[...]


def _classifier(focal, wc1, bc1, wc2, bc2):
    return pl.pallas_call(
        _cls_body,
        out_shape=jax.ShapeDtypeStruct((focal.shape[0], 2), _f32),
    )(focal, wc1, bc1.reshape(1, -1), wc2, bc2.reshape(1, -1))


def _attn_mat(a_s, a_d):
    a = jnp.zeros((4, 128, 16), _f32)
    for g in range(4):
        for j in range(2):
            hh = 2 * g + j
            a = a.at[g, j * 64:(j + 1) * 64, j].set(a_s[hh])
            a = a.at[g, j * 64:(j + 1) * 64, 2 + j].set(a_d[hh])
    return a


def kernel(x, edge_index, gene_idx, W1, a1s, a1d, b1, W2, a2s, a2d, b2,
           W3, a3s, a3d, b3, Wc1, bc1, Wc2, bc2):
    loop = jnp.arange(NN, dtype=edge_index.dtype)
    pad = jnp.zeros((EPAD - ETOT,), edge_index.dtype)
    srcp = jnp.concatenate([edge_index[0], loop, pad])
    dstp = jnp.concatenate([edge_index[1], loop, pad])
    zer12 = jnp.zeros((TPT, ACC12), _f32)
    zer3 = jnp.zeros((TPT, ACC3), _f32)
    zb = jnp.zeros((1, 1, 128), _f32)

    a1g = _attn_mat(a1s, a1d)
    a2g = _attn_mat(a2s, a2d)
    a3g = jnp.zeros((1, 64, 16), _f32).at[0, :, 0].set(a3s[0]).at[0, :, 1].set(a3d[0])

    xwr1 = _mm_grouped(x.reshape(1, NN, 128), W1.reshape(1, 128, 512),
                       zb, False, 4, 128)
    tabs1 = _tables4(xwr1, a1g)
    msg1 = _sc_layer12(xwr1.reshape(4 * NN, 128), tabs1.reshape(4 * NN, 16),
                       srcp, dstp, zer12)

    xwr2 = _mm_grouped(msg1.reshape(4, NN, 128), W2.reshape(4, 128, 512),
                       b1.reshape(4, 1, 128), True, 4, 128)
    tabs2 = _tables4(xwr2, a2g)
    msg2 = _sc_layer12(xwr2.reshape(4 * NN, 128), tabs2.reshape(4 * NN, 16),
                       srcp, dstp, zer12)

    xw3 = _mm_grouped(msg2.reshape(4, NN, 128), W3.reshape(4, 128, 64),
                      b2.reshape(4, 1, 128), True, 1, 64)
    xw3 = xw3.reshape(NN, 64)
    tabs3 = _tables4(xw3.reshape(1, NN, 64), a3g)
    p3 = _sc_layer3(xw3, tabs3.reshape(NN, 16), srcp, dstp, zer3)

    focal = _sc_focal(p3, gene_idx, b3)
    return _classifier(focal, Wc1, bc1, Wc2, bc2)
